# baseline (speedup 1.0000x reference)
import functools

import jax
import jax.numpy as jnp
from jax import lax
from jax.experimental import pallas as pl
from jax.experimental.pallas import tpu as pltpu
from jax.experimental.pallas import tpu_sc as plsc

BATCH = 16384
D = 32
NC = 2
NS = 16
L = 16
NW = NC * NS
B_PER_W = BATCH // NW
CHUNK = 128
N_CHUNK = B_PER_W // CHUNK
GROUPS = CHUNK // L


def _bpr_body(uidx_h, iidx_h, utab_h, itab_h, out_h,
              uidx_v, iidx_v, utile, itile, out_v, sem_u, sem_i):
    c = lax.axis_index("c")
    s = lax.axis_index("s")
    wid = s * NC + c
    base = wid * B_PER_W

    pltpu.sync_copy(uidx_h.at[pl.ds(base, B_PER_W)], uidx_v)
    pltpu.sync_copy(iidx_h.at[pl.ds(base, B_PER_W)], iidx_v)

    lane = jnp.arange(L, dtype=jnp.int32)

    for j in range(N_CHUNK):
        @plsc.parallel_loop(0, GROUPS, unroll=2)
        def fire(g, j=j):
            vu = uidx_v[pl.ds(j * CHUNK + g * L, L)]
            vi = iidx_v[pl.ds(j * CHUNK + g * L, L)]
            for l in range(L):
                k = g * L + l
                pltpu.async_copy(
                    utab_h.at[pl.ds(vu[l], 1)], utile.at[pl.ds(k, 1)], sem_u)
                pltpu.async_copy(
                    itab_h.at[pl.ds(vi[l], 1)], itile.at[pl.ds(k, 1)], sem_i)

        def drain(k, carry):
            pltpu.make_async_copy(
                utab_h.at[pl.ds(0, 1)], utile.at[pl.ds(k, 1)], sem_u).wait()
            pltpu.make_async_copy(
                itab_h.at[pl.ds(0, 1)], itile.at[pl.ds(k, 1)], sem_i).wait()
            return carry

        lax.fori_loop(0, CHUNK, drain, 0)

        @plsc.parallel_loop(0, GROUPS, unroll=2)
        def group_body(g, j=j):
            acc = jnp.zeros((L,), jnp.float32)
            for l in range(L):
                r = g * L + l
                u0 = utile[r, pl.ds(0, L)]
                u1 = utile[r, pl.ds(L, L)]
                i0 = itile[r, pl.ds(0, L)]
                i1 = itile[r, pl.ds(L, L)]
                v = jnp.sum(u0 * i0 + u1 * i1)
                acc = jnp.where(lane == l, v, acc)
            out_v[pl.ds(j * CHUNK + g * L, L)] = acc

    pltpu.sync_copy(out_v, out_h.at[pl.ds(base, B_PER_W)])


_bpr_sc = functools.partial(
    pl.kernel,
    mesh=plsc.VectorSubcoreMesh(core_axis_name="c", subcore_axis_name="s"),
    out_type=jax.ShapeDtypeStruct((BATCH,), jnp.float32),
    compiler_params=pltpu.CompilerParams(
        needs_layout_passes=False, use_tc_tiling_on_sc=True),
    scratch_types=[
        pltpu.VMEM((B_PER_W,), jnp.int32),
        pltpu.VMEM((B_PER_W,), jnp.int32),
        pltpu.VMEM((CHUNK, D), jnp.float32),
        pltpu.VMEM((CHUNK, D), jnp.float32),
        pltpu.VMEM((B_PER_W,), jnp.float32),
        pltpu.SemaphoreType.DMA,
        pltpu.SemaphoreType.DMA,
    ],
)(_bpr_body)


@jax.jit
def kernel(user_indices, item_indices, user_table, item_table):
    return _bpr_sc(user_indices, item_indices, user_table, item_table)


# speedup vs baseline: 1.0021x; 1.0021x over previous
import functools

import jax
import jax.numpy as jnp
from jax import lax
from jax.experimental import pallas as pl
from jax.experimental.pallas import tpu as pltpu
from jax.experimental.pallas import tpu_sc as plsc

BATCH = 16384
D = 32
NC = 2
NS = 16
L = 16
NW = NC * NS
B_PER_W = BATCH // NW
CHUNK = 128
N_CHUNK = B_PER_W // CHUNK
GROUPS = CHUNK // L


def _bpr_body(uidx_h, iidx_h, utab_h, itab_h, out_h,
              uidx_v, iidx_v, utile, itile, out_v, sem_u, sem_i):
    c = lax.axis_index("c")
    s = lax.axis_index("s")
    wid = s * NC + c
    base = wid * B_PER_W

    pltpu.sync_copy(uidx_h.at[pl.ds(base, B_PER_W)], uidx_v)
    pltpu.sync_copy(iidx_h.at[pl.ds(base, B_PER_W)], iidx_v)

    lane = jnp.arange(L, dtype=jnp.int32)

    for j in range(N_CHUNK):
        @plsc.parallel_loop(0, GROUPS, unroll=2)
        def fire(g, j=j):
            vu = uidx_v[pl.ds(j * CHUNK + g * L, L)]
            vi = iidx_v[pl.ds(j * CHUNK + g * L, L)]
            for l in range(L):
                k = g * L + l
                pltpu.async_copy(
                    utab_h.at[pl.ds(vu[l], 1)], utile.at[pl.ds(k, 1)], sem_u)
                pltpu.async_copy(
                    itab_h.at[pl.ds(vi[l], 1)], itile.at[pl.ds(k, 1)], sem_i)

        def drain(k, carry):
            pltpu.make_async_copy(
                utab_h.at[pl.ds(0, 1)], utile.at[pl.ds(k, 1)], sem_u).wait()
            pltpu.make_async_copy(
                itab_h.at[pl.ds(0, 1)], itile.at[pl.ds(k, 1)], sem_i).wait()
            return carry

        lax.fori_loop(0, CHUNK, drain, 0)

        @plsc.parallel_loop(0, GROUPS, unroll=2)
        def group_body(g, j=j):
            acc = jnp.zeros((L,), jnp.float32)
            for l in range(L):
                r = g * L + l
                u0 = utile[r, pl.ds(0, L)]
                u1 = utile[r, pl.ds(L, L)]
                i0 = itile[r, pl.ds(0, L)]
                i1 = itile[r, pl.ds(L, L)]
                v = jnp.sum(u0 * i0 + u1 * i1)
                acc = jnp.where(lane == l, v, acc)
            out_v[pl.ds(j * CHUNK + g * L, L)] = acc

    pltpu.sync_copy(out_v, out_h.at[pl.ds(base, B_PER_W)])


_bpr_sc = functools.partial(
    pl.kernel,
    mesh=plsc.VectorSubcoreMesh(core_axis_name="c", subcore_axis_name="s"),
    out_type=jax.ShapeDtypeStruct((BATCH,), jnp.float32),
    compiler_params=pltpu.CompilerParams(
        needs_layout_passes=False, use_tc_tiling_on_sc=True,
        skip_device_barrier=True),
    scratch_types=[
        pltpu.VMEM((B_PER_W,), jnp.int32),
        pltpu.VMEM((B_PER_W,), jnp.int32),
        pltpu.VMEM((CHUNK, D), jnp.float32),
        pltpu.VMEM((CHUNK, D), jnp.float32),
        pltpu.VMEM((B_PER_W,), jnp.float32),
        pltpu.SemaphoreType.DMA,
        pltpu.SemaphoreType.DMA,
    ],
)(_bpr_body)


@jax.jit
def kernel(user_indices, item_indices, user_table, item_table):
    return _bpr_sc(user_indices, item_indices, user_table, item_table)


# speedup vs baseline: 2.2440x; 2.2393x over previous
import functools

import jax
import jax.numpy as jnp
from jax import lax
from jax.experimental import pallas as pl
from jax.experimental.pallas import tpu as pltpu
from jax.experimental.pallas import tpu_sc as plsc

BATCH = 16384
D = 32
NC = 2
NS = 16
L = 16
NW = NC * NS
B_PER_W = BATCH // NW
CHUNK = 128
N_CHUNK = B_PER_W // CHUNK
GROUPS = CHUNK // L
SUB = 8


def _bpr_body(uidx_h, iidx_h, utab_h, itab_h, out_h,
              uidx_v, iidx_v, ublk, iblk, out_v, sem_u, sem_i):
    c = lax.axis_index("c")
    s = lax.axis_index("s")
    wid = s * NC + c
    base = wid * B_PER_W

    pltpu.sync_copy(uidx_h.at[pl.ds(base, B_PER_W)], uidx_v)
    pltpu.sync_copy(iidx_h.at[pl.ds(base, B_PER_W)], iidx_v)

    d_lo = jnp.arange(L, dtype=jnp.int32)
    d_hi = d_lo + L
    lane = d_lo

    for j in range(N_CHUNK):
        def group_body(g, carry, j=j):
            vu = uidx_v[pl.ds(j * CHUNK + g * L, L)]
            vi = iidx_v[pl.ds(j * CHUNK + g * L, L)]
            acc = jnp.zeros((L,), jnp.float32)
            for h in range(L // SUB):
                for l in range(SUB):
                    e = h * SUB + l
                    cbu = pl.multiple_of((vu[e] >> 7) << 7, 128)
                    cbi = pl.multiple_of((vi[e] >> 7) << 7, 128)
                    pltpu.async_copy(
                        utab_h.at[:, pl.ds(cbu, 128)], ublk.at[l], sem_u)
                    pltpu.async_copy(
                        itab_h.at[:, pl.ds(cbi, 128)], iblk.at[l], sem_i)
                for l in range(SUB):
                    pltpu.make_async_copy(
                        utab_h.at[:, pl.ds(0, 128)], ublk.at[l], sem_u).wait()
                    pltpu.make_async_copy(
                        itab_h.at[:, pl.ds(0, 128)], iblk.at[l], sem_i).wait()
                for l in range(SUB):
                    e = h * SUB + l
                    lf = jnp.full((L,), l, jnp.int32)
                    cu = jnp.full((L,), vu[e] & 127, jnp.int32)
                    ci = jnp.full((L,), vi[e] & 127, jnp.int32)
                    u0 = plsc.load_gather(ublk, [lf, d_lo, cu])
                    u1 = plsc.load_gather(ublk, [lf, d_hi, cu])
                    i0 = plsc.load_gather(iblk, [lf, d_lo, ci])
                    i1 = plsc.load_gather(iblk, [lf, d_hi, ci])
                    v = jnp.sum(u0 * i0 + u1 * i1)
                    acc = jnp.where(lane == e, v, acc)
            out_v[pl.ds(j * CHUNK + g * L, L)] = acc
            return carry

        lax.fori_loop(0, GROUPS, group_body, 0)

    pltpu.sync_copy(out_v, out_h.at[pl.ds(base, B_PER_W)])


_bpr_sc = functools.partial(
    pl.kernel,
    mesh=plsc.VectorSubcoreMesh(core_axis_name="c", subcore_axis_name="s"),
    out_type=jax.ShapeDtypeStruct((BATCH,), jnp.float32),
    compiler_params=pltpu.CompilerParams(
        needs_layout_passes=False, use_tc_tiling_on_sc=True),
    scratch_types=[
        pltpu.VMEM((B_PER_W,), jnp.int32),
        pltpu.VMEM((B_PER_W,), jnp.int32),
        pltpu.VMEM((SUB, D, 128), jnp.float32),
        pltpu.VMEM((SUB, D, 128), jnp.float32),
        pltpu.VMEM((B_PER_W,), jnp.float32),
        pltpu.SemaphoreType.DMA,
        pltpu.SemaphoreType.DMA,
    ],
)(_bpr_body)


@jax.jit
def kernel(user_indices, item_indices, user_table, item_table):
    return _bpr_sc(user_indices, item_indices, user_table.T, item_table.T)
